# baseline (speedup 1.0000x reference)
import functools

import jax
import jax.numpy as jnp
from jax import lax
from jax.experimental import pallas as pl
from jax.experimental.pallas import tpu as pltpu
from jax.experimental.pallas import tpu_sc as plsc

RES = 128.0
TABLE_MAX = 2 ** 19 - 1
N = 1_000_000
D = 8
L = 16

_INFO = plsc.get_sparse_core_info()
NC, NS = _INFO.num_cores, _INFO.num_subcores
NW = NC * NS
PER_W = (-(-N // NW) + 7) // 8 * 8
C = 2048
NCHUNK = -(-PER_W // C)

_MESH = plsc.VectorSubcoreMesh(core_axis_name="c", subcore_axis_name="s")


@functools.partial(
    pl.kernel,
    mesh=_MESH,
    compiler_params=pltpu.CompilerParams(
        needs_layout_passes=False, use_tc_tiling_on_sc=False
    ),
    out_type=jax.ShapeDtypeStruct((N, D), jnp.float32),
    scratch_types=[
        pltpu.VMEM((C * 3,), jnp.float32),
        pltpu.VMEM((C,), jnp.int32),
        pltpu.VMEM((C, D), jnp.float32),
        pltpu.SemaphoreType.DMA,
    ],
)
def _hash_gather(pos_hbm, table_hbm, out_hbm, pos_v, idx_v, rows_v, sem):
    wid = lax.axis_index("s") * NC + lax.axis_index("c")
    tile_base = jnp.minimum(wid * PER_W, N - PER_W)
    lane3 = lax.iota(jnp.int32, L) * 3

    def chunk_body(ci, _):
        base = tile_base + jnp.minimum(ci * C, PER_W - C)
        pltpu.sync_copy(pos_hbm.at[pl.ds(base * 3, C * 3)], pos_v)

        def grp(j, _):
            fidx = j * (3 * L) + lane3
            x = plsc.load_gather(pos_v, [fidx])
            y = plsc.load_gather(pos_v, [fidx + 1])
            z = plsc.load_gather(pos_v, [fidx + 2])
            xs = jnp.clip((x + 1.0) * 0.5 * RES, 0.0, RES - 1.0)
            ys = jnp.clip((y + 1.0) * 0.5 * RES, 0.0, RES - 1.0)
            zs = jnp.clip((z + 1.0) * 0.5 * RES, 0.0, RES - 1.0)
            f = xs * (RES * RES) + ys * RES + zs
            idx = jnp.clip(f.astype(jnp.int32), 0, TABLE_MAX)
            idx_v[pl.ds(j * L, L)] = idx
            return _

        lax.fori_loop(0, C // L, grp, None)
        pltpu.async_copy(table_hbm.at[idx_v], rows_v, sem).wait()
        pltpu.sync_copy(rows_v, out_hbm.at[pl.ds(base, C)])
        return _

    lax.fori_loop(0, NCHUNK, chunk_body, None)


def kernel(positions, table):
    return _hash_gather(positions.reshape(-1), table)
